# baseline (speedup 1.0000x reference)
import functools
import jax
import jax.numpy as jnp
from jax import lax
from jax.experimental import pallas as pl
from jax.experimental.pallas import tpu as pltpu
from jax.experimental.pallas import tpu_sc as plsc

_N = 10000
_E = 320000
_EH = _E // 2
_D = 128
_OD = 128
_ED = 16
_EB = 3200
_NBH = _EH // _EB
_NC, _NS = 2, 16
_NW = _NC * _NS
_NCHUNK = 125
_NP = 10240
_STRIPE = _NP // _NS
_NB_ROWS = 2000

_sc_mesh = plsc.VectorSubcoreMesh(core_axis_name="c", subcore_axis_name="s")


def _full(shape):
    nd = len(shape)
    return pl.BlockSpec(shape, lambda i: (0,) * nd)


def _eblk(width, blk0=0):
    return pl.BlockSpec((_EB, width), lambda i: (i + blk0, 0))


def _f32(shape):
    return jax.ShapeDtypeStruct(shape, jnp.float32)


def _ln(x, g, b, eps=1e-6):
    m = jnp.mean(x, axis=-1, keepdims=True)
    v = jnp.mean((x - m) * (x - m), axis=-1, keepdims=True)
    return (x - m) * jax.lax.rsqrt(v + eps) * g + b


def _sc_gather_pair_body(e0, epw, k, ta_hbm, tb_hbm, ia_hbm, ib_hbm, o_hbm,
                         idxa0, idxb0, idxa1, idxb1,
                         rowsa0, rowsb0, rowsa1, rowsb1,
                         semi0, semi1, semg0, semg1, semw0, semw1):
    wid = lax.axis_index("s") * _NC + lax.axis_index("c")
    base = wid * epw
    idxa = (idxa0, idxa1)
    idxb = (idxb0, idxb1)
    rowsa = (rowsa0, rowsa1)
    rowsb = (rowsb0, rowsb1)
    semi = (semi0, semi1)
    semg = (semg0, semg1)
    semw = (semw0, semw1)

    def off(c):
        return base + c * k

    def issue_i(c, b):
        pltpu.async_copy(ia_hbm.at[pl.ds(e0 + off(c), k)], idxa[b], semi[b])
        pltpu.async_copy(ib_hbm.at[pl.ds(e0 + off(c), k)], idxb[b], semi[b])

    def wait_i(c, b):
        pltpu.make_async_copy(ia_hbm.at[pl.ds(e0 + off(c), k)], idxa[b],
                              semi[b]).wait()
        pltpu.make_async_copy(ib_hbm.at[pl.ds(e0 + off(c), k)], idxb[b],
                              semi[b]).wait()

    def issue_g(b):
        pltpu.async_copy(ta_hbm.at[idxa[b]], rowsa[b], semg[b])
        pltpu.async_copy(tb_hbm.at[idxb[b]], rowsb[b], semg[b])

    def wait_g(b):
        pltpu.make_async_copy(ta_hbm.at[idxa[b]], rowsa[b], semg[b]).wait()
        pltpu.make_async_copy(tb_hbm.at[idxb[b]], rowsb[b], semg[b]).wait()

    def issue_w(c, b):
        pltpu.async_copy(rowsa[b], o_hbm.at[pl.ds(off(c), k)], semw[b])

    def wait_w(c, b):
        pltpu.make_async_copy(rowsa[b], o_hbm.at[pl.ds(off(c), k)],
                              semw[b]).wait()

    def splice(b):
        ra = rowsa[b]
        rb = rowsb[b]

        @pl.loop(0, k, step=4)
        def _(j):
            for jj in range(4):
                for kk in range(_OD // 32):
                    sl = pl.ds(_OD // 2 + 16 * kk, 16)
                    ra[j + jj, sl] = rb[j + jj, sl]

    issue_i(0, 0)
    wait_i(0, 0)
    issue_g(0)
    issue_i(1, 1)
    wait_i(1, 1)
    issue_g(1)
    wait_g(0)
    splice(0)
    issue_w(0, 0)
    issue_i(2, 0)

    @pl.loop(0, (_NCHUNK - 3) // 2)
    def _(g):
        for b, d in ((0, 2), (1, 3)):
            c = 2 * g + d
            wait_i(c, b)
            wait_w(c - 2, b)
            issue_g(b)
            wait_g(1 - b)
            issue_i(c + 1, 1 - b)
            splice(1 - b)
            issue_w(c - 1, 1 - b)

    c = _NCHUNK - 1
    wait_i(c, 0)
    wait_w(c - 2, 0)
    issue_g(0)
    wait_g(1)
    splice(1)
    issue_w(c - 1, 1)
    wait_g(0)
    splice(0)
    issue_w(c, 0)
    wait_w(c - 1, 1)
    wait_w(c, 0)


def _make_gather(e0, k):
    epw = k * _NCHUNK
    ne = epw * _NW
    return functools.partial(
        pl.kernel,
        mesh=_sc_mesh,
        out_type=_f32((ne, _OD)),
        scratch_types=[
            pltpu.VMEM((k,), jnp.int32),
            pltpu.VMEM((k,), jnp.int32),
            pltpu.VMEM((k,), jnp.int32),
            pltpu.VMEM((k,), jnp.int32),
            pltpu.VMEM((k, _OD), jnp.float32),
            pltpu.VMEM((k, _OD), jnp.float32),
            pltpu.VMEM((k, _OD), jnp.float32),
            pltpu.VMEM((k, _OD), jnp.float32),
            pltpu.SemaphoreType.DMA,
            pltpu.SemaphoreType.DMA,
            pltpu.SemaphoreType.DMA,
            pltpu.SemaphoreType.DMA,
            pltpu.SemaphoreType.DMA,
            pltpu.SemaphoreType.DMA,
        ],
    )(functools.partial(_sc_gather_pair_body, e0, epw, k))


def _sc_scatter_body(e0, epw, k, m_hbm, dst_hbm, zeros_hbm, out_hbm,
                     idx0, idx1, rows0, rows1,
                     seml0, seml1, sems0, sems1, acc):
    cid = lax.axis_index("c")
    sid = lax.axis_index("s")
    wid = sid * _NC + cid
    base = wid * epw
    idx = (idx0, idx1)
    rows = (rows0, rows1)
    seml = (seml0, seml1)
    sems = (sems0, sems1)

    pltpu.sync_copy(zeros_hbm.at[pl.ds(sid * _STRIPE, _STRIPE)],
                    acc.at[pl.ds(sid * _STRIPE, _STRIPE)])
    plsc.subcore_barrier()

    def off(c):
        return base + c * k

    def issue_l(c, b):
        pltpu.async_copy(dst_hbm.at[pl.ds(e0 + off(c), k)], idx[b], seml[b])
        pltpu.async_copy(m_hbm.at[pl.ds(off(c), k)], rows[b], seml[b])

    def wait_l(c, b):
        pltpu.make_async_copy(dst_hbm.at[pl.ds(e0 + off(c), k)], idx[b],
                              seml[b]).wait()
        pltpu.make_async_copy(m_hbm.at[pl.ds(off(c), k)], rows[b],
                              seml[b]).wait()

    def issue_s(b):
        pltpu.async_copy(rows[b], acc.at[idx[b]], sems[b], add=True)

    def wait_s(b):
        pltpu.make_async_copy(rows[b], acc.at[idx[b]], sems[b]).wait()

    issue_l(0, 0)
    wait_l(0, 0)
    issue_s(0)
    issue_l(1, 1)

    @pl.loop(0, (_NCHUNK - 1) // 2)
    def _(g):
        for b, d in ((1, 1), (0, 2)):
            c = 2 * g + d
            wait_l(c, b)
            issue_s(b)
            wait_s(1 - b)

            @pl.when(c + 1 < _NCHUNK)
            def _():
                issue_l(c + 1, 1 - b)

    wait_s(0)

    plsc.subcore_barrier()
    pltpu.sync_copy(acc.at[pl.ds(sid * _STRIPE, _STRIPE)],
                    out_hbm.at[pl.ds(cid * _NP + sid * _STRIPE, _STRIPE)])


def _make_scatter(e0, k):
    epw = k * _NCHUNK
    return functools.partial(
        pl.kernel,
        mesh=_sc_mesh,
        out_type=_f32((_NC * _NP, _OD)),
        scratch_types=[
            pltpu.VMEM((k,), jnp.int32),
            pltpu.VMEM((k,), jnp.int32),
            pltpu.VMEM((k, _OD), jnp.float32),
            pltpu.VMEM((k, _OD), jnp.float32),
            pltpu.SemaphoreType.DMA,
            pltpu.SemaphoreType.DMA,
            pltpu.SemaphoreType.DMA,
            pltpu.SemaphoreType.DMA,
            pltpu.VMEM_SHARED((_NP, _OD), jnp.float32),
        ],
    )(functools.partial(_sc_scatter_body, e0, epw, k))


def _node_pre_body(x_ref, dummy_ref, g0_ref, b0_ref, projw_ref, projb_ref,
                   gatew_ref, gateb_ref, w1ds_ref,
                   pdps_ref, skip_ref, gate_ref):
    x = x_ref[...]
    invalid = x[:, 0:1] == -999.0
    x = jnp.where(invalid, dummy_ref[...], x)
    x = _ln(x, g0_ref[...], b0_ref[...])
    skip = x @ projw_ref[...] + projb_ref[...]
    gate = jax.nn.sigmoid(skip @ gatew_ref[...] + gateb_ref[...])
    skip_ref[...] = skip
    gate_ref[...] = gate
    pdps_ref[...] = x @ w1ds_ref[...]


def _edge_enc_body(ea_ref,
                   lng_ref, lnb_ref, w1_ref, b1_ref, w2_ref, b2_ref,
                   w3_ref, b3_ref, cw1_ref, cb1_ref, cw2_ref, cb2_ref,
                   eenc_ref):
    ea = ea_ref[...]
    h = _ln(ea, lng_ref[...], lnb_ref[...])
    h = jnp.maximum(h @ w1_ref[...] + b1_ref[...], 0.0)
    h = jnp.maximum(h @ w2_ref[...] + b2_ref[...], 0.0)
    eenc = h @ w3_ref[...] + b3_ref[...]
    ew = jnp.maximum(ea @ cw1_ref[...] + cb1_ref[...], 0.0)
    ew = jax.nn.sigmoid(ew @ cw2_ref[...] + cb2_ref[...])
    eenc_ref[...] = eenc * ew


def _conv1_body(g1_ref, eenc_ref, e1w_ref, e1b_ref,
                w2_ref, b2_ref, w3_ref, b3_ref, m1_ref):
    a1 = eenc_ref[...] @ e1w_ref[...] + e1b_ref[...]
    h = jnp.maximum(g1_ref[:, :_OD // 2] + g1_ref[:, _OD // 2:] + a1, 0.0)
    h = jnp.maximum(h @ w2_ref[...] + b2_ref[...], 0.0)
    m1 = h @ w3_ref[...] + b3_ref[...]
    m1_ref[...] = jnp.concatenate(
        [m1, jnp.zeros((_EB, _OD // 2 - 1), jnp.float32),
         jnp.ones((_EB, 1), jnp.float32)], axis=1)


def _conv2_body(eenc_ref, g2_ref, wds_ref,
                we_ref, b1_ref, w2_ref, b2_ref,
                w3_ref, b3_ref, m2_ref):
    h = jnp.maximum(g2_ref[...] @ wds_ref[...]
                    + eenc_ref[...] @ we_ref[...] + b1_ref[...], 0.0)
    h = jnp.maximum(h @ w2_ref[...] + b2_ref[...], 0.0)
    m2_ref[...] = h @ w3_ref[...] + b3_ref[...]


def _node_mid_body(s1pa_ref, s1pb_ref, g_ref, b_ref,
                   t2_ref, cnt_ref):
    def tot(ref, c0, c1):
        return ref[0, :, c0:c1] + ref[1, :, c0:c1]
    cnt = jnp.maximum(tot(s1pa_ref, _OD - 1, _OD)
                      + tot(s1pb_ref, _OD - 1, _OD), 1.0)
    cnt_ref[...] = cnt
    s1 = tot(s1pa_ref, 0, _OD // 2) + tot(s1pb_ref, 0, _OD // 2)
    x1 = s1 / cnt
    x1 = _ln(x1, g_ref[...], b_ref[...])
    x1 = jnp.where(x1 > 0, x1, 0.01 * x1)
    t2_ref[...] = jnp.concatenate([x1, x1], axis=1)


def _node_fin_body(s2pa_ref, s2pb_ref, cnt_ref, efsp_ref, skip_ref, gate_ref,
                   g2_ref, b2_ref, w1_ref, bb1_ref, w2_ref, bb2_ref,
                   w3_ref, bb3_ref, xfc_ref, probs_ref):
    cnt = cnt_ref[...]
    s2 = (s2pa_ref[0, :, :] + s2pa_ref[1, :, :]
          + s2pb_ref[0, :, :] + s2pb_ref[1, :, :])
    x2 = _ln(s2 / cnt, g2_ref[...], b2_ref[...])
    x2 = jnp.maximum(x2, 0.0)
    gate = gate_ref[...]
    xf = gate * skip_ref[...] + (1.0 - gate) * x2
    efm = (efsp_ref[0, :, :] + efsp_ref[1, :, :]) / cnt
    xfc = jnp.concatenate([xf, efm], axis=1)
    xfc_ref[...] = xfc
    h = xfc @ w1_ref[...] + bb1_ref[...]
    h = jnp.where(h > 0, h, jnp.exp(jnp.minimum(h, 0.0)) - 1.0)
    h = h @ w2_ref[...] + bb2_ref[...]
    h = jnp.where(h > 0, h, jnp.exp(jnp.minimum(h, 0.0)) - 1.0)
    probs_ref[...] = h @ w3_ref[...] + bb3_ref[...]


@jax.jit
def _run(x_in, edge_index, edge_attr, params):
    p = params
    x = x_in[0]
    ea = edge_attr[0]
    src = edge_index[0, 0]
    dst = edge_index[0, 1]
    r = lambda b: b.reshape(1, -1)

    c1w1 = p['c1_W1']
    w1ds = jnp.concatenate([c1w1[:, :_D].T, c1w1[:, _D:2 * _D].T], axis=1)
    w1e = c1w1[:, 2 * _D:].T
    c2w1 = p['c2_W1']
    w2ds = jnp.concatenate([c2w1[:, :_OD // 2].T,
                            c2w1[:, _OD // 2:_OD].T], axis=0)
    w2e = c2w1[:, _OD:].T

    zeros128 = jnp.zeros((_NP, _OD), jnp.float32)

    gather_a = _make_gather(0, 40)
    gather_b = _make_gather(_EH, 40)
    scatter_a = _make_scatter(0, 40)
    scatter_b = _make_scatter(_EH, 40)
    scatter_full = _make_scatter(0, 80)

    pdps, skip, gate = pl.pallas_call(
        _node_pre_body,
        out_shape=(_f32((_N, _OD)), _f32((_N, _OD)), _f32((_N, _OD))),
        in_specs=[_full((_N, _D)), _full((1, _D)), _full((1, _D)),
                  _full((1, _D)), _full((_D, _OD)), _full((1, _OD)),
                  _full((_OD, _OD)), _full((1, _OD)),
                  _full((_D, _OD))],
        out_specs=(_full((_N, _OD)), _full((_N, _OD)), _full((_N, _OD))),
        grid=(1,),
    )(x, r(p['dummy']), r(p['bn0_g']), r(p['bn0_b']),
      p['proj_W'].T, r(p['proj_b']), p['gate_W'].T, r(p['gate_b']),
      w1ds)

    g1a = gather_a(pdps, pdps, dst, src)
    g1b = gather_b(pdps, pdps, dst, src)

    eenc = pl.pallas_call(
        _edge_enc_body,
        out_shape=_f32((_E, _OD)),
        in_specs=[_eblk(_ED),
                  _full((1, _ED)), _full((1, _ED)),
                  _full((_ED, _OD)), _full((1, _OD)),
                  _full((_OD, 2 * _OD)), _full((1, 2 * _OD)),
                  _full((2 * _OD, _OD)), _full((1, _OD)),
                  _full((_ED, _ED)), _full((1, _ED)),
                  _full((_ED, 1)), _full((1, 1))],
        out_specs=_eblk(_OD),
        grid=(2 * _NBH,),
    )(ea, r(p['ee_ln_g']), r(p['ee_ln_b']),
      p['ee_W1'].T, r(p['ee_b1']), p['ee_W2'].T, r(p['ee_b2']),
      p['ee_W3'].T, r(p['ee_b3']),
      p['ec_W1'].T, r(p['ec_b1']), p['ec_W2'].T, r(p['ec_b2']))

    def conv1(g1, blk0):
        return pl.pallas_call(
            _conv1_body,
            out_shape=_f32((_EH, _OD)),
            in_specs=[_eblk(_OD), _eblk(_OD, blk0),
                      _full((_OD, _OD // 2)), _full((1, _OD // 2)),
                      _full((_OD // 2, _OD // 2)), _full((1, _OD // 2)),
                      _full((_OD // 2, _OD // 2)), _full((1, _OD // 2))],
            out_specs=_eblk(_OD),
            grid=(_NBH,),
        )(g1, eenc, w1e, r(p['c1_b1']),
          p['c1_W2'].T, r(p['c1_b2']), p['c1_W3'].T, r(p['c1_b3']))

    m1a = conv1(g1a, 0)
    s1pa = scatter_a(m1a, dst, zeros128)
    m1b = conv1(g1b, _NBH)
    s1pb = scatter_b(m1b, dst, zeros128)

    _pblk = pl.BlockSpec((_NC, _NB_ROWS, _OD), lambda i: (0, i, 0))
    _nblk = lambda w: pl.BlockSpec((_NB_ROWS, w), lambda i: (i, 0))
    t2, cnt = pl.pallas_call(
        _node_mid_body,
        out_shape=(_f32((_N, _OD)), _f32((_N, 1))),
        in_specs=[_pblk, _pblk,
                  _full((1, _OD // 2)), _full((1, _OD // 2))],
        out_specs=(_nblk(_OD), _nblk(1)),
        grid=(_N // _NB_ROWS,),
    )(s1pa.reshape(_NC, _NP, _OD), s1pb.reshape(_NC, _NP, _OD),
      r(p['bn1_g']), r(p['bn1_b']))

    g2a = gather_a(t2, t2, dst, src)
    g2b = gather_b(t2, t2, dst, src)

    efsp = scatter_full(eenc, dst, zeros128)

    def conv2(g2, blk0):
        return pl.pallas_call(
            _conv2_body,
            out_shape=_f32((_EH, _OD)),
            in_specs=[_eblk(_OD, blk0), _eblk(_OD),
                      _full((_OD, _OD)),
                      _full((_OD, _OD)), _full((1, _OD)),
                      _full((_OD, _OD)), _full((1, _OD)),
                      _full((_OD, _OD)), _full((1, _OD))],
            out_specs=_eblk(_OD),
            grid=(_NBH,),
        )(eenc, g2, w2ds,
          w2e, r(p['c2_b1']),
          p['c2_W2'].T, r(p['c2_b2']), p['c2_W3'].T, r(p['c2_b3']))

    m2a = conv2(g2a, 0)
    s2pa = scatter_a(m2a, dst, zeros128)
    m2b = conv2(g2b, _NBH)
    s2pb = scatter_b(m2b, dst, zeros128)

    xfc, probs = pl.pallas_call(
        _node_fin_body,
        out_shape=(_f32((_N, 2 * _OD)), _f32((_N, 1))),
        in_specs=[_pblk, _pblk,
                  _nblk(1),
                  _pblk,
                  _nblk(_OD), _nblk(_OD),
                  _full((1, _OD)), _full((1, _OD)),
                  _full((2 * _OD, _OD)), _full((1, _OD)),
                  _full((_OD, _OD // 2)), _full((1, _OD // 2)),
                  _full((_OD // 2, 1)), _full((1, 1))],
        out_specs=(_nblk(2 * _OD), _nblk(1)),
        grid=(_N // _NB_ROWS,),
    )(s2pa.reshape(_NC, _NP, _OD), s2pb.reshape(_NC, _NP, _OD),
      cnt, efsp.reshape(_NC, _NP, _OD), skip, gate,
      r(p['bn2_g']), r(p['bn2_b']),
      p['np_W1'].T, r(p['np_b1']), p['np_W2'].T, r(p['np_b2']),
      p['np_W3'].T, r(p['np_b3']))

    return xfc, probs


def kernel(x_in, edge_index, edge_attr, params):
    return _run(x_in, edge_index, edge_attr, params)


# speedup vs baseline: 1.1447x; 1.1447x over previous
import functools
import jax
import jax.numpy as jnp
from jax import lax
from jax.experimental import pallas as pl
from jax.experimental.pallas import tpu as pltpu
from jax.experimental.pallas import tpu_sc as plsc

_N = 10000
_E = 320000
_EA = 153600
_EBB = _E - _EA
_D = 128
_OD = 128
_ED = 16
_EB = 3200
_NBA = _EA // _EB
_NBB = _EBB // _EB
_NC, _NS = 2, 16
_NW = _NC * _NS
_NP = 10240
_STRIPE = _NP // _NS
_NB_ROWS = 2000

_sc_mesh = plsc.VectorSubcoreMesh(core_axis_name="c", subcore_axis_name="s")


def _full(shape):
    nd = len(shape)
    return pl.BlockSpec(shape, lambda i: (0,) * nd)


def _eblk(width, blk0=0):
    return pl.BlockSpec((_EB, width), lambda i: (i + blk0, 0))


def _f32(shape):
    return jax.ShapeDtypeStruct(shape, jnp.float32)


def _ln(x, g, b, eps=1e-6):
    m = jnp.mean(x, axis=-1, keepdims=True)
    v = jnp.mean((x - m) * (x - m), axis=-1, keepdims=True)
    return (x - m) * jax.lax.rsqrt(v + eps) * g + b


def _sc_gather_pair_body(e0, epw, k, n, ta_hbm, tb_hbm, ia_hbm, ib_hbm, o_hbm,
                         idxa0, idxb0, idxa1, idxb1,
                         rowsa0, rowsb0, rowsa1, rowsb1,
                         semi0, semi1, semg0, semg1, semw0, semw1):
    wid = lax.axis_index("s") * _NC + lax.axis_index("c")
    base = wid * epw
    idxa = (idxa0, idxa1)
    idxb = (idxb0, idxb1)
    rowsa = (rowsa0, rowsa1)
    rowsb = (rowsb0, rowsb1)
    semi = (semi0, semi1)
    semg = (semg0, semg1)
    semw = (semw0, semw1)

    def off(c):
        return base + c * k

    def issue_i(c, b):
        pltpu.async_copy(ia_hbm.at[pl.ds(e0 + off(c), k)], idxa[b], semi[b])
        pltpu.async_copy(ib_hbm.at[pl.ds(e0 + off(c), k)], idxb[b], semi[b])

    def wait_i(c, b):
        pltpu.make_async_copy(ia_hbm.at[pl.ds(e0 + off(c), k)], idxa[b],
                              semi[b]).wait()
        pltpu.make_async_copy(ib_hbm.at[pl.ds(e0 + off(c), k)], idxb[b],
                              semi[b]).wait()

    def issue_g(b):
        pltpu.async_copy(ta_hbm.at[idxa[b]], rowsa[b], semg[b])
        pltpu.async_copy(tb_hbm.at[idxb[b]], rowsb[b], semg[b])

    def wait_g(b):
        pltpu.make_async_copy(ta_hbm.at[idxa[b]], rowsa[b], semg[b]).wait()
        pltpu.make_async_copy(tb_hbm.at[idxb[b]], rowsb[b], semg[b]).wait()

    def issue_w(c, b):
        pltpu.async_copy(rowsa[b], o_hbm.at[pl.ds(off(c), k)], semw[b])

    def wait_w(c, b):
        pltpu.make_async_copy(rowsa[b], o_hbm.at[pl.ds(off(c), k)],
                              semw[b]).wait()

    def splice(b):
        ra = rowsa[b]
        rb = rowsb[b]

        @pl.loop(0, k, step=4)
        def _(j):
            for jj in range(4):
                for kk in range(_OD // 32):
                    sl = pl.ds(_OD // 2 + 16 * kk, 16)
                    ra[j + jj, sl] = rb[j + jj, sl]

    issue_i(0, 0)
    wait_i(0, 0)
    issue_g(0)
    issue_i(1, 1)
    wait_i(1, 1)
    issue_g(1)
    wait_g(0)
    splice(0)
    issue_w(0, 0)
    issue_i(2, 0)

    pairs = (n - 3) // 2

    @pl.loop(0, pairs)
    def _(g):
        for b, d in ((0, 2), (1, 3)):
            c = 2 * g + d
            wait_i(c, b)
            wait_w(c - 2, b)
            issue_g(b)
            wait_g(1 - b)
            issue_i(c + 1, 1 - b)
            splice(1 - b)
            issue_w(c - 1, 1 - b)

    for c in range(2 + 2 * pairs, n):
        b = c % 2
        wait_i(c, b)
        wait_w(c - 2, b)
        issue_g(b)
        wait_g(1 - b)
        if c + 1 < n:
            issue_i(c + 1, 1 - b)
        splice(1 - b)
        issue_w(c - 1, 1 - b)
    lb = (n - 1) % 2
    wait_g(lb)
    splice(lb)
    issue_w(n - 1, lb)
    wait_w(n - 2, 1 - lb)
    wait_w(n - 1, lb)


def _make_gather(e0, k, n):
    epw = k * n
    ne = epw * _NW
    return functools.partial(
        pl.kernel,
        mesh=_sc_mesh,
        out_type=_f32((ne, _OD)),
        scratch_types=[
            pltpu.VMEM((k,), jnp.int32),
            pltpu.VMEM((k,), jnp.int32),
            pltpu.VMEM((k,), jnp.int32),
            pltpu.VMEM((k,), jnp.int32),
            pltpu.VMEM((k, _OD), jnp.float32),
            pltpu.VMEM((k, _OD), jnp.float32),
            pltpu.VMEM((k, _OD), jnp.float32),
            pltpu.VMEM((k, _OD), jnp.float32),
            pltpu.SemaphoreType.DMA,
            pltpu.SemaphoreType.DMA,
            pltpu.SemaphoreType.DMA,
            pltpu.SemaphoreType.DMA,
            pltpu.SemaphoreType.DMA,
            pltpu.SemaphoreType.DMA,
        ],
    )(functools.partial(_sc_gather_pair_body, e0, epw, k, n))


def _sc_scatter_body(e0, epw, k, n, m_hbm, dst_hbm, zeros_hbm, out_hbm,
                     idx0, idx1, rows0, rows1,
                     seml0, seml1, sems0, sems1, acc):
    cid = lax.axis_index("c")
    sid = lax.axis_index("s")
    wid = sid * _NC + cid
    base = wid * epw
    idx = (idx0, idx1)
    rows = (rows0, rows1)
    seml = (seml0, seml1)
    sems = (sems0, sems1)

    pltpu.sync_copy(zeros_hbm.at[pl.ds(sid * _STRIPE, _STRIPE)],
                    acc.at[pl.ds(sid * _STRIPE, _STRIPE)])
    plsc.subcore_barrier()

    def off(c):
        return base + c * k

    def issue_l(c, b):
        pltpu.async_copy(dst_hbm.at[pl.ds(e0 + off(c), k)], idx[b], seml[b])
        pltpu.async_copy(m_hbm.at[pl.ds(off(c), k)], rows[b], seml[b])

    def wait_l(c, b):
        pltpu.make_async_copy(dst_hbm.at[pl.ds(e0 + off(c), k)], idx[b],
                              seml[b]).wait()
        pltpu.make_async_copy(m_hbm.at[pl.ds(off(c), k)], rows[b],
                              seml[b]).wait()

    def issue_s(b):
        pltpu.async_copy(rows[b], acc.at[idx[b]], sems[b], add=True)

    def wait_s(b):
        pltpu.make_async_copy(rows[b], acc.at[idx[b]], sems[b]).wait()

    issue_l(0, 0)
    wait_l(0, 0)
    issue_s(0)
    issue_l(1, 1)

    pairs = (n - 1) // 2

    @pl.loop(0, pairs)
    def _(g):
        for b, d in ((1, 1), (0, 2)):
            c = 2 * g + d
            wait_l(c, b)
            issue_s(b)
            wait_s(1 - b)

            @pl.when(c + 1 < n)
            def _():
                issue_l(c + 1, 1 - b)

    for c in range(1 + 2 * pairs, n):
        b = c % 2
        wait_l(c, b)
        issue_s(b)
        wait_s(1 - b)
        if c + 1 < n:
            issue_l(c + 1, 1 - b)
    wait_s((n - 1) % 2)

    plsc.subcore_barrier()
    pltpu.sync_copy(acc.at[pl.ds(sid * _STRIPE, _STRIPE)],
                    out_hbm.at[pl.ds(cid * _NP + sid * _STRIPE, _STRIPE)])


def _make_scatter(e0, k, n):
    epw = k * n
    return functools.partial(
        pl.kernel,
        mesh=_sc_mesh,
        out_type=_f32((_NC * _NP, _OD)),
        scratch_types=[
            pltpu.VMEM((k,), jnp.int32),
            pltpu.VMEM((k,), jnp.int32),
            pltpu.VMEM((k, _OD), jnp.float32),
            pltpu.VMEM((k, _OD), jnp.float32),
            pltpu.SemaphoreType.DMA,
            pltpu.SemaphoreType.DMA,
            pltpu.SemaphoreType.DMA,
            pltpu.SemaphoreType.DMA,
            pltpu.VMEM_SHARED((_NP, _OD), jnp.float32),
        ],
    )(functools.partial(_sc_scatter_body, e0, epw, k, n))


def _node_pre_body(x_ref, dummy_ref, g0_ref, b0_ref, projw_ref, projb_ref,
                   gatew_ref, gateb_ref, w1ds_ref,
                   pdps_ref, skip_ref, gate_ref):
    x = x_ref[...]
    invalid = x[:, 0:1] == -999.0
    x = jnp.where(invalid, dummy_ref[...], x)
    x = _ln(x, g0_ref[...], b0_ref[...])
    skip = x @ projw_ref[...] + projb_ref[...]
    gate = jax.nn.sigmoid(skip @ gatew_ref[...] + gateb_ref[...])
    skip_ref[...] = skip
    gate_ref[...] = gate
    pdps_ref[...] = x @ w1ds_ref[...]


def _edge_enc_body(ea_ref,
                   lng_ref, lnb_ref, w1_ref, b1_ref, w2_ref, b2_ref,
                   w3_ref, b3_ref, cw1_ref, cb1_ref, cw2_ref, cb2_ref,
                   eenc_ref):
    ea = ea_ref[...]
    h = _ln(ea, lng_ref[...], lnb_ref[...])
    h = jnp.maximum(h @ w1_ref[...] + b1_ref[...], 0.0)
    h = jnp.maximum(h @ w2_ref[...] + b2_ref[...], 0.0)
    eenc = h @ w3_ref[...] + b3_ref[...]
    ew = jnp.maximum(ea @ cw1_ref[...] + cb1_ref[...], 0.0)
    ew = jax.nn.sigmoid(ew @ cw2_ref[...] + cb2_ref[...])
    eenc_ref[...] = eenc * ew


def _conv1_body(g1_ref, eenc_ref, e1w_ref, e1b_ref,
                w2_ref, b2_ref, w3_ref, b3_ref, m1_ref):
    a1 = eenc_ref[...] @ e1w_ref[...] + e1b_ref[...]
    h = jnp.maximum(g1_ref[:, :_OD // 2] + g1_ref[:, _OD // 2:] + a1, 0.0)
    h = jnp.maximum(h @ w2_ref[...] + b2_ref[...], 0.0)
    m1 = h @ w3_ref[...] + b3_ref[...]
    m1_ref[...] = jnp.concatenate(
        [m1, jnp.zeros((_EB, _OD // 2 - 1), jnp.float32),
         jnp.ones((_EB, 1), jnp.float32)], axis=1)


def _conv2_body(eenc_ref, g2_ref, wds_ref,
                we_ref, b1_ref, w2_ref, b2_ref,
                w3_ref, b3_ref, m2_ref):
    h = jnp.maximum(g2_ref[...] @ wds_ref[...]
                    + eenc_ref[...] @ we_ref[...] + b1_ref[...], 0.0)
    h = jnp.maximum(h @ w2_ref[...] + b2_ref[...], 0.0)
    m2_ref[...] = h @ w3_ref[...] + b3_ref[...]


def _node_mid_body(s1pa_ref, s1pb_ref, g_ref, b_ref,
                   t2_ref, cnt_ref):
    def tot(ref, c0, c1):
        return ref[0, :, c0:c1] + ref[1, :, c0:c1]
    cnt = jnp.maximum(tot(s1pa_ref, _OD - 1, _OD)
                      + tot(s1pb_ref, _OD - 1, _OD), 1.0)
    cnt_ref[...] = cnt
    s1 = tot(s1pa_ref, 0, _OD // 2) + tot(s1pb_ref, 0, _OD // 2)
    x1 = s1 / cnt
    x1 = _ln(x1, g_ref[...], b_ref[...])
    x1 = jnp.where(x1 > 0, x1, 0.01 * x1)
    t2_ref[...] = jnp.concatenate([x1, x1], axis=1)


def _node_fin_body(s2pa_ref, s2pb_ref, cnt_ref, efsp_ref, skip_ref, gate_ref,
                   g2_ref, b2_ref, w1_ref, bb1_ref, w2_ref, bb2_ref,
                   w3_ref, bb3_ref, xfc_ref, probs_ref):
    cnt = cnt_ref[...]
    s2 = (s2pa_ref[0, :, :] + s2pa_ref[1, :, :]
          + s2pb_ref[0, :, :] + s2pb_ref[1, :, :])
    x2 = _ln(s2 / cnt, g2_ref[...], b2_ref[...])
    x2 = jnp.maximum(x2, 0.0)
    gate = gate_ref[...]
    xf = gate * skip_ref[...] + (1.0 - gate) * x2
    efm = (efsp_ref[0, :, :] + efsp_ref[1, :, :]) / cnt
    xfc = jnp.concatenate([xf, efm], axis=1)
    xfc_ref[...] = xfc
    h = xfc @ w1_ref[...] + bb1_ref[...]
    h = jnp.where(h > 0, h, jnp.exp(jnp.minimum(h, 0.0)) - 1.0)
    h = h @ w2_ref[...] + bb2_ref[...]
    h = jnp.where(h > 0, h, jnp.exp(jnp.minimum(h, 0.0)) - 1.0)
    probs_ref[...] = h @ w3_ref[...] + bb3_ref[...]


@jax.jit
def _run(x_in, edge_index, edge_attr, params):
    p = params
    x = x_in[0]
    ea = edge_attr[0]
    src = edge_index[0, 0]
    dst = edge_index[0, 1]
    r = lambda b: b.reshape(1, -1)

    c1w1 = p['c1_W1']
    w1ds = jnp.concatenate([c1w1[:, :_D].T, c1w1[:, _D:2 * _D].T], axis=1)
    w1e = c1w1[:, 2 * _D:].T
    c2w1 = p['c2_W1']
    w2ds = jnp.concatenate([c2w1[:, :_OD // 2].T,
                            c2w1[:, _OD // 2:_OD].T], axis=0)
    w2e = c2w1[:, _OD:].T

    zeros128 = jnp.zeros((_NP, _OD), jnp.float32)

    gather_a = _make_gather(0, 80, 60)
    gather_b = _make_gather(_EA, 80, 65)
    scatter_a = _make_scatter(0, 80, 60)
    scatter_b = _make_scatter(_EA, 80, 65)
    scatter_full = _make_scatter(0, 80, 125)

    pdps, skip, gate = pl.pallas_call(
        _node_pre_body,
        out_shape=(_f32((_N, _OD)), _f32((_N, _OD)), _f32((_N, _OD))),
        in_specs=[_full((_N, _D)), _full((1, _D)), _full((1, _D)),
                  _full((1, _D)), _full((_D, _OD)), _full((1, _OD)),
                  _full((_OD, _OD)), _full((1, _OD)),
                  _full((_D, _OD))],
        out_specs=(_full((_N, _OD)), _full((_N, _OD)), _full((_N, _OD))),
        grid=(1,),
    )(x, r(p['dummy']), r(p['bn0_g']), r(p['bn0_b']),
      p['proj_W'].T, r(p['proj_b']), p['gate_W'].T, r(p['gate_b']),
      w1ds)

    g1a = gather_a(pdps, pdps, dst, src)
    g1b = gather_b(pdps, pdps, dst, src)

    eenc = pl.pallas_call(
        _edge_enc_body,
        out_shape=_f32((_E, _OD)),
        in_specs=[_eblk(_ED),
                  _full((1, _ED)), _full((1, _ED)),
                  _full((_ED, _OD)), _full((1, _OD)),
                  _full((_OD, 2 * _OD)), _full((1, 2 * _OD)),
                  _full((2 * _OD, _OD)), _full((1, _OD)),
                  _full((_ED, _ED)), _full((1, _ED)),
                  _full((_ED, 1)), _full((1, 1))],
        out_specs=_eblk(_OD),
        grid=(_NBA + _NBB,),
    )(ea, r(p['ee_ln_g']), r(p['ee_ln_b']),
      p['ee_W1'].T, r(p['ee_b1']), p['ee_W2'].T, r(p['ee_b2']),
      p['ee_W3'].T, r(p['ee_b3']),
      p['ec_W1'].T, r(p['ec_b1']), p['ec_W2'].T, r(p['ec_b2']))

    def conv1(g1, nb, blk0):
        return pl.pallas_call(
            _conv1_body,
            out_shape=_f32((nb * _EB, _OD)),
            in_specs=[_eblk(_OD), _eblk(_OD, blk0),
                      _full((_OD, _OD // 2)), _full((1, _OD // 2)),
                      _full((_OD // 2, _OD // 2)), _full((1, _OD // 2)),
                      _full((_OD // 2, _OD // 2)), _full((1, _OD // 2))],
            out_specs=_eblk(_OD),
            grid=(nb,),
        )(g1, eenc, w1e, r(p['c1_b1']),
          p['c1_W2'].T, r(p['c1_b2']), p['c1_W3'].T, r(p['c1_b3']))

    m1a = conv1(g1a, _NBA, 0)
    s1pa = scatter_a(m1a, dst, zeros128)
    m1b = conv1(g1b, _NBB, _NBA)
    s1pb = scatter_b(m1b, dst, zeros128)

    _pblk = pl.BlockSpec((_NC, _NB_ROWS, _OD), lambda i: (0, i, 0))
    _nblk = lambda w: pl.BlockSpec((_NB_ROWS, w), lambda i: (i, 0))
    t2, cnt = pl.pallas_call(
        _node_mid_body,
        out_shape=(_f32((_N, _OD)), _f32((_N, 1))),
        in_specs=[_pblk, _pblk,
                  _full((1, _OD // 2)), _full((1, _OD // 2))],
        out_specs=(_nblk(_OD), _nblk(1)),
        grid=(_N // _NB_ROWS,),
    )(s1pa.reshape(_NC, _NP, _OD), s1pb.reshape(_NC, _NP, _OD),
      r(p['bn1_g']), r(p['bn1_b']))

    g2a = gather_a(t2, t2, dst, src)
    g2b = gather_b(t2, t2, dst, src)

    efsp = scatter_full(eenc, dst, zeros128)

    def conv2(g2, nb, blk0):
        return pl.pallas_call(
            _conv2_body,
            out_shape=_f32((nb * _EB, _OD)),
            in_specs=[_eblk(_OD, blk0), _eblk(_OD),
                      _full((_OD, _OD)),
                      _full((_OD, _OD)), _full((1, _OD)),
                      _full((_OD, _OD)), _full((1, _OD)),
                      _full((_OD, _OD)), _full((1, _OD))],
            out_specs=_eblk(_OD),
            grid=(nb,),
        )(eenc, g2, w2ds,
          w2e, r(p['c2_b1']),
          p['c2_W2'].T, r(p['c2_b2']), p['c2_W3'].T, r(p['c2_b3']))

    m2a = conv2(g2a, _NBA, 0)
    s2pa = scatter_a(m2a, dst, zeros128)
    m2b = conv2(g2b, _NBB, _NBA)
    s2pb = scatter_b(m2b, dst, zeros128)

    xfc, probs = pl.pallas_call(
        _node_fin_body,
        out_shape=(_f32((_N, 2 * _OD)), _f32((_N, 1))),
        in_specs=[_pblk, _pblk,
                  _nblk(1),
                  _pblk,
                  _nblk(_OD), _nblk(_OD),
                  _full((1, _OD)), _full((1, _OD)),
                  _full((2 * _OD, _OD)), _full((1, _OD)),
                  _full((_OD, _OD // 2)), _full((1, _OD // 2)),
                  _full((_OD // 2, 1)), _full((1, 1))],
        out_specs=(_nblk(2 * _OD), _nblk(1)),
        grid=(_N // _NB_ROWS,),
    )(s2pa.reshape(_NC, _NP, _OD), s2pb.reshape(_NC, _NP, _OD),
      cnt, efsp.reshape(_NC, _NP, _OD), skip, gate,
      r(p['bn2_g']), r(p['bn2_b']),
      p['np_W1'].T, r(p['np_b1']), p['np_W2'].T, r(p['np_b2']),
      p['np_W3'].T, r(p['np_b3']))

    return xfc, probs


def kernel(x_in, edge_index, edge_attr, params):
    return _run(x_in, edge_index, edge_attr, params)


# speedup vs baseline: 1.1537x; 1.0079x over previous
import functools
import jax
import jax.numpy as jnp
from jax import lax
from jax.experimental import pallas as pl
from jax.experimental.pallas import tpu as pltpu
from jax.experimental.pallas import tpu_sc as plsc

_N = 10000
_E = 320000
_D = 128
_OD = 128
_ED = 16
_EB = 3200
_NC, _NS = 2, 16
_NW = _NC * _NS
_EPW = _E // _NW
_K = 80
_NCHUNK = _EPW // _K
_NP = 10240
_STRIPE = _NP // _NS

_sc_mesh = plsc.VectorSubcoreMesh(core_axis_name="c", subcore_axis_name="s")


def _full(shape):
    nd = len(shape)
    return pl.BlockSpec(shape, lambda i: (0,) * nd)


def _eblk(width):
    return pl.BlockSpec((_EB, width), lambda i: (i, 0))


def _f32(shape):
    return jax.ShapeDtypeStruct(shape, jnp.float32)


def _ln(x, g, b, eps=1e-6):
    m = jnp.mean(x, axis=-1, keepdims=True)
    v = jnp.mean((x - m) * (x - m), axis=-1, keepdims=True)
    return (x - m) * jax.lax.rsqrt(v + eps) * g + b


def _sc_gather_pair_body(ta_hbm, tb_hbm, ia_hbm, ib_hbm, o_hbm,
                         idxa0, idxb0, idxa1, idxb1,
                         rowsa0, rowsb0, rowsa1, rowsb1,
                         semi0, semi1, semg0, semg1, semw0, semw1):
    wid = lax.axis_index("s") * _NC + lax.axis_index("c")
    base = wid * _EPW
    idxa = (idxa0, idxa1)
    idxb = (idxb0, idxb1)
    rowsa = (rowsa0, rowsa1)
    rowsb = (rowsb0, rowsb1)
    semi = (semi0, semi1)
    semg = (semg0, semg1)
    semw = (semw0, semw1)

    def off(c):
        return base + c * _K

    def issue_i(c, b):
        pltpu.async_copy(ia_hbm.at[pl.ds(off(c), _K)], idxa[b], semi[b])
        pltpu.async_copy(ib_hbm.at[pl.ds(off(c), _K)], idxb[b], semi[b])

    def wait_i(c, b):
        pltpu.make_async_copy(ia_hbm.at[pl.ds(off(c), _K)], idxa[b],
                              semi[b]).wait()
        pltpu.make_async_copy(ib_hbm.at[pl.ds(off(c), _K)], idxb[b],
                              semi[b]).wait()

    def issue_g(b):
        pltpu.async_copy(ta_hbm.at[idxa[b]], rowsa[b], semg[b])
        pltpu.async_copy(tb_hbm.at[idxb[b]], rowsb[b], semg[b])

    def wait_g(b):
        pltpu.make_async_copy(ta_hbm.at[idxa[b]], rowsa[b], semg[b]).wait()
        pltpu.make_async_copy(tb_hbm.at[idxb[b]], rowsb[b], semg[b]).wait()

    def issue_w(c, b):
        pltpu.async_copy(rowsa[b], o_hbm.at[pl.ds(off(c), _K)], semw[b])

    def wait_w(c, b):
        pltpu.make_async_copy(rowsa[b], o_hbm.at[pl.ds(off(c), _K)],
                              semw[b]).wait()

    def splice(b):
        ra = rowsa[b]
        rb = rowsb[b]

        @pl.loop(0, _K, step=4)
        def _(j):
            for jj in range(4):
                for k in range(_OD // 32):
                    sl = pl.ds(_OD // 2 + 16 * k, 16)
                    ra[j + jj, sl] = rb[j + jj, sl]

    issue_i(0, 0)
    wait_i(0, 0)
    issue_g(0)
    issue_i(1, 1)
    wait_i(1, 1)
    issue_g(1)
    wait_g(0)
    splice(0)
    issue_w(0, 0)
    issue_i(2, 0)

    @pl.loop(0, (_NCHUNK - 3) // 2)
    def _(g):
        for b, d in ((0, 2), (1, 3)):
            c = 2 * g + d
            wait_i(c, b)
            wait_w(c - 2, b)
            issue_g(b)
            wait_g(1 - b)
            issue_i(c + 1, 1 - b)
            splice(1 - b)
            issue_w(c - 1, 1 - b)

    c = _NCHUNK - 1
    wait_i(c, 0)
    wait_w(c - 2, 0)
    issue_g(0)
    wait_g(1)
    splice(1)
    issue_w(c - 1, 1)
    wait_g(0)
    splice(0)
    issue_w(c, 0)
    wait_w(c - 1, 1)
    wait_w(c, 0)


_gather_pair = functools.partial(
    pl.kernel,
    mesh=_sc_mesh,
    out_type=_f32((_E, _OD)),
    scratch_types=[
        pltpu.VMEM((_K,), jnp.int32),
        pltpu.VMEM((_K,), jnp.int32),
        pltpu.VMEM((_K,), jnp.int32),
        pltpu.VMEM((_K,), jnp.int32),
        pltpu.VMEM((_K, _OD), jnp.float32),
        pltpu.VMEM((_K, _OD), jnp.float32),
        pltpu.VMEM((_K, _OD), jnp.float32),
        pltpu.VMEM((_K, _OD), jnp.float32),
        pltpu.SemaphoreType.DMA,
        pltpu.SemaphoreType.DMA,
        pltpu.SemaphoreType.DMA,
        pltpu.SemaphoreType.DMA,
        pltpu.SemaphoreType.DMA,
        pltpu.SemaphoreType.DMA,
    ],
)(_sc_gather_pair_body)


def _sc_scatter_body(m_hbm, dst_hbm, zeros_hbm, out_hbm,
                     idx0, idx1, rows0, rows1,
                     seml0, seml1, sems0, sems1, acc):
    cid = lax.axis_index("c")
    sid = lax.axis_index("s")
    wid = sid * _NC + cid
    base = wid * _EPW
    idx = (idx0, idx1)
    rows = (rows0, rows1)
    seml = (seml0, seml1)
    sems = (sems0, sems1)

    pltpu.sync_copy(zeros_hbm.at[pl.ds(sid * _STRIPE, _STRIPE)],
                    acc.at[pl.ds(sid * _STRIPE, _STRIPE)])
    plsc.subcore_barrier()

    def off(c):
        return base + c * _K

    def issue_l(c, b):
        pltpu.async_copy(dst_hbm.at[pl.ds(off(c), _K)], idx[b], seml[b])
        pltpu.async_copy(m_hbm.at[pl.ds(off(c), _K)], rows[b], seml[b])

    def wait_l(c, b):
        pltpu.make_async_copy(dst_hbm.at[pl.ds(off(c), _K)], idx[b],
                              seml[b]).wait()
        pltpu.make_async_copy(m_hbm.at[pl.ds(off(c), _K)], rows[b],
                              seml[b]).wait()

    def issue_s(b):
        pltpu.async_copy(rows[b], acc.at[idx[b]], sems[b], add=True)

    def wait_s(b):
        pltpu.make_async_copy(rows[b], acc.at[idx[b]], sems[b]).wait()

    issue_l(0, 0)
    wait_l(0, 0)
    issue_s(0)
    issue_l(1, 1)

    @pl.loop(0, (_NCHUNK - 1) // 2)
    def _(g):
        for b, d in ((1, 1), (0, 2)):
            c = 2 * g + d
            wait_l(c, b)
            issue_s(b)
            wait_s(1 - b)

            @pl.when(c + 1 < _NCHUNK)
            def _():
                issue_l(c + 1, 1 - b)

    wait_s(0)

    plsc.subcore_barrier()
    pltpu.sync_copy(acc.at[pl.ds(sid * _STRIPE, _STRIPE)],
                    out_hbm.at[pl.ds(cid * _NP + sid * _STRIPE, _STRIPE)])


_scatter = functools.partial(
    pl.kernel,
    mesh=_sc_mesh,
    out_type=_f32((_NC * _NP, _OD)),
    scratch_types=[
        pltpu.VMEM((_K,), jnp.int32),
        pltpu.VMEM((_K,), jnp.int32),
        pltpu.VMEM((_K, _OD), jnp.float32),
        pltpu.VMEM((_K, _OD), jnp.float32),
        pltpu.SemaphoreType.DMA,
        pltpu.SemaphoreType.DMA,
        pltpu.SemaphoreType.DMA,
        pltpu.SemaphoreType.DMA,
        pltpu.VMEM_SHARED((_NP, _OD), jnp.float32),
    ],
)(_sc_scatter_body)


def _node_pre_body(x_ref, dummy_ref, g0_ref, b0_ref, projw_ref, projb_ref,
                   gatew_ref, gateb_ref, w1ds_ref,
                   pdps_ref, skip_ref, gate_ref):
    x = x_ref[...]
    invalid = x[:, 0:1] == -999.0
    x = jnp.where(invalid, dummy_ref[...], x)
    x = _ln(x, g0_ref[...], b0_ref[...])
    skip = x @ projw_ref[...] + projb_ref[...]
    gate = jax.nn.sigmoid(skip @ gatew_ref[...] + gateb_ref[...])
    skip_ref[...] = skip
    gate_ref[...] = gate
    pdps_ref[...] = x @ w1ds_ref[...]


def _edge_enc_body(ea_ref,
                   lng_ref, lnb_ref, w1_ref, b1_ref, w2_ref, b2_ref,
                   w3_ref, b3_ref, cw1_ref, cb1_ref, cw2_ref, cb2_ref,
                   eenc_ref):
    ea = ea_ref[...]
    h = _ln(ea, lng_ref[...], lnb_ref[...])
    h = jnp.maximum(h @ w1_ref[...] + b1_ref[...], 0.0)
    h = jnp.maximum(h @ w2_ref[...] + b2_ref[...], 0.0)
    eenc = h @ w3_ref[...] + b3_ref[...]
    ew = jnp.maximum(ea @ cw1_ref[...] + cb1_ref[...], 0.0)
    ew = jax.nn.sigmoid(ew @ cw2_ref[...] + cb2_ref[...])
    eenc_ref[...] = eenc * ew


def _conv1_body(g1_ref, eenc_ref, e1w_ref, e1b_ref,
                w2_ref, b2_ref, w3_ref, b3_ref, m1_ref):
    a1 = eenc_ref[...] @ e1w_ref[...] + e1b_ref[...]
    h = jnp.maximum(g1_ref[:, :_OD // 2] + g1_ref[:, _OD // 2:] + a1, 0.0)
    h = jnp.maximum(h @ w2_ref[...] + b2_ref[...], 0.0)
    m1 = h @ w3_ref[...] + b3_ref[...]
    m1_ref[...] = jnp.concatenate(
        [m1, jnp.zeros((_EB, _OD // 2 - 1), jnp.float32),
         jnp.ones((_EB, 1), jnp.float32)], axis=1)


def _conv2_body(eenc_ref, g2_ref, wds_ref,
                we_ref, b1_ref, w2_ref, b2_ref,
                w3_ref, b3_ref, m2_ref):
    h = jnp.maximum(g2_ref[...] @ wds_ref[...]
                    + eenc_ref[...] @ we_ref[...] + b1_ref[...], 0.0)
    h = jnp.maximum(h @ w2_ref[...] + b2_ref[...], 0.0)
    m2_ref[...] = h @ w3_ref[...] + b3_ref[...]


def _node_mid_body(s1p_ref, g_ref, b_ref,
                   t2_ref, cnt_ref):
    cnt = jnp.maximum(s1p_ref[0:_N, _OD - 1:_OD]
                      + s1p_ref[_NP:_NP + _N, _OD - 1:_OD], 1.0)
    cnt_ref[...] = cnt
    s1 = s1p_ref[0:_N, :_OD // 2] + s1p_ref[_NP:_NP + _N, :_OD // 2]
    x1 = s1 / cnt
    x1 = _ln(x1, g_ref[...], b_ref[...])
    x1 = jnp.where(x1 > 0, x1, 0.01 * x1)
    t2_ref[...] = jnp.concatenate([x1, x1], axis=1)


def _node_fin_body(s2p_ref, cnt_ref, efsp_ref, skip_ref, gate_ref,
                   g2_ref, b2_ref, w1_ref, bb1_ref, w2_ref, bb2_ref,
                   w3_ref, bb3_ref, xfc_ref, probs_ref):
    cnt = cnt_ref[...]
    s2 = s2p_ref[0:_N, :] + s2p_ref[_NP:_NP + _N, :]
    x2 = _ln(s2 / cnt, g2_ref[...], b2_ref[...])
    x2 = jnp.maximum(x2, 0.0)
    gate = gate_ref[...]
    xf = gate * skip_ref[...] + (1.0 - gate) * x2
    efm = (efsp_ref[0:_N, :] + efsp_ref[_NP:_NP + _N, :]) / cnt
    xfc = jnp.concatenate([xf, efm], axis=1)
    xfc_ref[...] = xfc
    h = xfc @ w1_ref[...] + bb1_ref[...]
    h = jnp.where(h > 0, h, jnp.exp(jnp.minimum(h, 0.0)) - 1.0)
    h = h @ w2_ref[...] + bb2_ref[...]
    h = jnp.where(h > 0, h, jnp.exp(jnp.minimum(h, 0.0)) - 1.0)
    probs_ref[...] = h @ w3_ref[...] + bb3_ref[...]


@jax.jit
def _run(x_in, edge_index, edge_attr, params):
    p = params
    x = x_in[0]
    ea = edge_attr[0]
    src = edge_index[0, 0]
    dst = edge_index[0, 1]
    r = lambda b: b.reshape(1, -1)

    c1w1 = p['c1_W1']
    w1ds = jnp.concatenate([c1w1[:, :_D].T, c1w1[:, _D:2 * _D].T], axis=1)
    w1e = c1w1[:, 2 * _D:].T
    c2w1 = p['c2_W1']
    w2ds = jnp.concatenate([c2w1[:, :_OD // 2].T,
                            c2w1[:, _OD // 2:_OD].T], axis=0)
    w2e = c2w1[:, _OD:].T

    zeros128 = jnp.zeros((_NP, _OD), jnp.float32)

    pdps, skip, gate = pl.pallas_call(
        _node_pre_body,
        out_shape=(_f32((_N, _OD)), _f32((_N, _OD)), _f32((_N, _OD))),
        in_specs=[_full((_N, _D)), _full((1, _D)), _full((1, _D)),
                  _full((1, _D)), _full((_D, _OD)), _full((1, _OD)),
                  _full((_OD, _OD)), _full((1, _OD)),
                  _full((_D, _OD))],
        out_specs=(_full((_N, _OD)), _full((_N, _OD)), _full((_N, _OD))),
        grid=(1,),
    )(x, r(p['dummy']), r(p['bn0_g']), r(p['bn0_b']),
      p['proj_W'].T, r(p['proj_b']), p['gate_W'].T, r(p['gate_b']),
      w1ds)

    nblk = _E // _EB
    eenc = pl.pallas_call(
        _edge_enc_body,
        out_shape=_f32((_E, _OD)),
        in_specs=[_eblk(_ED),
                  _full((1, _ED)), _full((1, _ED)),
                  _full((_ED, _OD)), _full((1, _OD)),
                  _full((_OD, 2 * _OD)), _full((1, 2 * _OD)),
                  _full((2 * _OD, _OD)), _full((1, _OD)),
                  _full((_ED, _ED)), _full((1, _ED)),
                  _full((_ED, 1)), _full((1, 1))],
        out_specs=_eblk(_OD),
        grid=(nblk,),
    )(ea, r(p['ee_ln_g']), r(p['ee_ln_b']),
      p['ee_W1'].T, r(p['ee_b1']), p['ee_W2'].T, r(p['ee_b2']),
      p['ee_W3'].T, r(p['ee_b3']),
      p['ec_W1'].T, r(p['ec_b1']), p['ec_W2'].T, r(p['ec_b2']))

    g1 = _gather_pair(pdps, pdps, dst, src)

    m1 = pl.pallas_call(
        _conv1_body,
        out_shape=_f32((_E, _OD)),
        in_specs=[_eblk(_OD), _eblk(_OD),
                  _full((_OD, _OD // 2)), _full((1, _OD // 2)),
                  _full((_OD // 2, _OD // 2)), _full((1, _OD // 2)),
                  _full((_OD // 2, _OD // 2)), _full((1, _OD // 2))],
        out_specs=_eblk(_OD),
        grid=(nblk,),
    )(g1, eenc, w1e, r(p['c1_b1']),
      p['c1_W2'].T, r(p['c1_b2']), p['c1_W3'].T, r(p['c1_b3']))

    s1p = _scatter(m1, dst, zeros128)

    t2, cnt = pl.pallas_call(
        _node_mid_body,
        out_shape=(_f32((_N, _OD)), _f32((_N, 1))),
        in_specs=[_full((_NC * _NP, _OD)),
                  _full((1, _OD // 2)), _full((1, _OD // 2))],
        out_specs=(_full((_N, _OD)), _full((_N, 1))),
        grid=(1,),
    )(s1p, r(p['bn1_g']), r(p['bn1_b']))

    g2 = _gather_pair(t2, t2, dst, src)

    efsp = _scatter(eenc, dst, zeros128)

    m2 = pl.pallas_call(
        _conv2_body,
        out_shape=_f32((_E, _OD)),
        in_specs=[_eblk(_OD), _eblk(_OD),
                  _full((_OD, _OD)),
                  _full((_OD, _OD)), _full((1, _OD)),
                  _full((_OD, _OD)), _full((1, _OD)),
                  _full((_OD, _OD)), _full((1, _OD))],
        out_specs=_eblk(_OD),
        grid=(nblk,),
    )(eenc, g2, w2ds,
      w2e, r(p['c2_b1']),
      p['c2_W2'].T, r(p['c2_b2']), p['c2_W3'].T, r(p['c2_b3']))

    s2p = _scatter(m2, dst, zeros128)

    xfc, probs = pl.pallas_call(
        _node_fin_body,
        out_shape=(_f32((_N, 2 * _OD)), _f32((_N, 1))),
        in_specs=[_full((_NC * _NP, _OD)), _full((_N, 1)),
                  _full((_NC * _NP, _OD)),
                  _full((_N, _OD)), _full((_N, _OD)),
                  _full((1, _OD)), _full((1, _OD)),
                  _full((2 * _OD, _OD)), _full((1, _OD)),
                  _full((_OD, _OD // 2)), _full((1, _OD // 2)),
                  _full((_OD // 2, 1)), _full((1, 1))],
        out_specs=(_full((_N, 2 * _OD)), _full((_N, 1))),
        grid=(1,),
    )(s2p, cnt, efsp, skip, gate,
      r(p['bn2_g']), r(p['bn2_b']),
      p['np_W1'].T, r(p['np_b1']), p['np_W2'].T, r(p['np_b2']),
      p['np_W3'].T, r(p['np_b3']))

    return xfc, probs


def kernel(x_in, edge_index, edge_attr, params):
    return _run(x_in, edge_index, edge_attr, params)
